# baseline (speedup 1.0000x reference)
import jax
import jax.numpy as jnp
from jax.experimental import pallas as pl

_NUM_CLASSES = 1000
_ROWS_PER_BLOCK = 512


def _onehot_block(x_ref, o_ref):
    idx = x_ref[0, 0, :]
    classes = jax.lax.broadcasted_iota(
        jnp.int32, (_ROWS_PER_BLOCK, _NUM_CLASSES), 1)
    o_ref[...] = (classes == idx[:, None]).astype(jnp.float32)


def kernel(x):
    n = x.shape[0] * x.shape[1]
    nb = n // _ROWS_PER_BLOCK
    xr = x.reshape(nb, 1, _ROWS_PER_BLOCK)
    out = pl.pallas_call(
        _onehot_block,
        grid=(nb,),
        in_specs=[pl.BlockSpec((1, 1, _ROWS_PER_BLOCK), lambda i: (i, 0, 0))],
        out_specs=pl.BlockSpec((_ROWS_PER_BLOCK, _NUM_CLASSES), lambda i: (i, 0)),
        out_shape=jax.ShapeDtypeStruct((n, _NUM_CLASSES), jnp.float32),
    )(xr)
    return out.reshape(x.shape[0], x.shape[1], _NUM_CLASSES)


# speedup vs baseline: 1.4563x; 1.4563x over previous
import jax
import jax.numpy as jnp
from jax.experimental import pallas as pl

_NUM_CLASSES = 1000
_BR = 64


def _onehot_block(x_ref, o_ref):
    idx = x_ref[...]
    classes = jax.lax.broadcasted_iota(
        jnp.int32, (_BR, idx.shape[1], _NUM_CLASSES), 2)
    o_ref[...] = (classes == idx[:, :, None]).astype(jnp.float32)


def kernel(x):
    n0, n1 = x.shape
    out = pl.pallas_call(
        _onehot_block,
        grid=(n0 // _BR,),
        in_specs=[pl.BlockSpec((_BR, n1), lambda i: (i, 0))],
        out_specs=pl.BlockSpec((_BR, n1, _NUM_CLASSES), lambda i: (i, 0, 0)),
        out_shape=jax.ShapeDtypeStruct((n0, n1, _NUM_CLASSES), jnp.float32),
    )(x)
    return out
